# baseline (speedup 1.0000x reference)
import functools

import jax
import jax.numpy as jnp
from jax import lax
from jax.experimental import pallas as pl
from jax.experimental.pallas import tpu as pltpu
from jax.experimental.pallas import tpu_sc as plsc

OUT = 7
SCALE = 6.5
BB, NN, CC, HH, WW = 2, 1000, 256, 128, 128
RR = BB * NN
RPAD = 2048
NBINS = OUT * OUT
NTOT = RPAD * NBINS
ROWS_PER_STEP = 32
GROWS = 4 * ROWS_PER_STEP
NWORKERS = 32
NSTEPS = NTOT // ROWS_PER_STEP
STEPS_PER_W = NSTEPS // NWORKERS
LANES = 16


def _coord_body(p_ref, idx_ref, w_ref):
    cx = p_ref[0:1, :]
    cy = p_ref[1:2, :]
    w = p_ref[2:3, :]
    h = p_ref[3:4, :]
    lane = lax.broadcasted_iota(jnp.int32, (1, RPAD), 1)
    in_range = (lane < RR).astype(jnp.float32)
    b = (lane >= NN).astype(jnp.int32)

    x1 = (cx - w * 0.5) * SCALE
    y1 = (cy - h * 0.5) * SCALE
    rsw = x1 - 0.5
    rsh = y1 - 0.5
    bin_w = (w * SCALE) / OUT
    bin_h = (h * SCALE) / OUT

    ii = lax.broadcasted_iota(jnp.int32, (NBINS, 1), 0)
    gi = (ii // OUT).astype(jnp.float32) + 0.5
    gj = (ii % OUT).astype(jnp.float32) + 0.5

    py = rsh + gi * bin_h
    px = rsw + gj * bin_w

    def interp(coord, size):
        valid = (coord >= -1.0) & (coord <= float(size))
        c = jnp.maximum(coord, 0.0)
        low = jnp.floor(c)
        cond = low >= float(size - 1)
        low = jnp.where(cond, float(size - 1), low)
        high = jnp.minimum(low + 1.0, float(size - 1))
        c = jnp.where(cond, float(size - 1), c)
        frac = c - low
        return low, high, frac, valid

    yl, yh, ly, vy = interp(py, HH)
    xl, xh, lx, vx = interp(px, WW)
    hy = 1.0 - ly
    hx = 1.0 - lx
    vf = (vy & vx).astype(jnp.float32) * in_range

    base = b * (HH * WW)
    yli = yl.astype(jnp.int32) * WW
    yhi = yh.astype(jnp.int32) * WW
    xli = xl.astype(jnp.int32)
    xhi = xh.astype(jnp.int32)

    idx_ref[...] = jnp.concatenate(
        [base + yli + xli, base + yli + xhi, base + yhi + xli, base + yhi + xhi],
        axis=0,
    )
    w_ref[...] = jnp.concatenate(
        [hy * hx * vf, hy * lx * vf, ly * hx * vf, ly * lx * vf], axis=0
    )


def _coords(proposals):
    prop = proposals.reshape(RR, 5).T
    propt = jnp.zeros((8, RPAD), jnp.float32).at[:5, :RR].set(prop)
    idx_all, w_all = pl.pallas_call(
        _coord_body,
        out_shape=(
            jax.ShapeDtypeStruct((4 * NBINS, RPAD), jnp.int32),
            jax.ShapeDtypeStruct((4 * NBINS, RPAD), jnp.float32),
        ),
    )(propt)
    idx4 = idx_all.reshape(4, NBINS, RPAD).transpose(2, 1, 0).reshape(NTOT * 4)
    w4 = w_all.reshape(4, NBINS, RPAD).transpose(2, 1, 0).reshape(NTOT, 4)
    return idx4, w4


def _sc_kernel(table, idx4):
    mesh = plsc.VectorSubcoreMesh(core_axis_name="c", subcore_axis_name="s")
    LAST = STEPS_PER_W - 1

    @functools.partial(
        pl.kernel,
        mesh=mesh,
        out_type=jax.ShapeDtypeStruct((NTOT * 4, CC), jnp.float32),
        scratch_types=[
            pltpu.VMEM((2, GROWS), jnp.int32),
            pltpu.VMEM((2, GROWS, CC), jnp.float32),
            pltpu.SemaphoreType.DMA((3, 2)),
        ],
    )
    def k(table_hbm, idx_hbm, out_hbm, idx_v, g_v, sems):
        wid = lax.axis_index("s") * 2 + lax.axis_index("c")
        base_step = wid * STEPS_PER_W

        def issue_idx(t, b):
            row0 = (base_step + t) * ROWS_PER_STEP
            pltpu.make_async_copy(
                idx_hbm.at[pl.ds(row0 * 4, GROWS)], idx_v.at[b], sems.at[0, b]
            ).start()

        def wait_idx(b):
            pltpu.make_async_copy(
                idx_hbm.at[pl.ds(0, GROWS)], idx_v.at[b], sems.at[0, b]
            ).wait()

        def issue_gather(b):
            pltpu.make_async_copy(
                table_hbm.at[idx_v.at[b]], g_v.at[b], sems.at[1, b]
            ).start()

        def wait_gather(b):
            pltpu.make_async_copy(
                table_hbm.at[idx_v.at[b]], g_v.at[b], sems.at[1, b]
            ).wait()

        def issue_store(t, b):
            row0 = (base_step + t) * ROWS_PER_STEP
            pltpu.make_async_copy(
                g_v.at[b], out_hbm.at[pl.ds(row0 * 4, GROWS)], sems.at[2, b]
            ).start()

        def wait_store(b):
            pltpu.make_async_copy(
                g_v.at[b], out_hbm.at[pl.ds(0, GROWS)], sems.at[2, b]
            ).wait()

        issue_idx(0, 0)
        issue_idx(1, 1)
        wait_idx(0)
        issue_gather(0)

        def body(t, b, nb):
            wait_gather(b)
            issue_store(t, b)

            @pl.when(t < LAST)
            def _():
                wait_idx(nb)

                @pl.when(t >= 1)
                def _():
                    wait_store(nb)

                issue_gather(nb)

            @pl.when(t + 2 <= LAST)
            def _():
                issue_idx(t + 2, b)

        @pl.loop(0, STEPS_PER_W // 2)
        def _(u):
            body(2 * u, 0, 1)
            body(2 * u + 1, 1, 0)

        wait_store(0)
        wait_store(1)

    return k(table, idx4)


TRB = 8


def _fma_body(g_ref, w_ref, out_ref):
    g3 = g_ref[...].reshape(TRB * NBINS, 4, CC)
    w = w_ref[...]
    val = (
        w[:, 0:1] * g3[:, 0, :]
        + w[:, 1:2] * g3[:, 1, :]
        + w[:, 2:3] * g3[:, 2, :]
        + w[:, 3:4] * g3[:, 3, :]
    )
    vb = val.astype(jnp.bfloat16).reshape(TRB, NBINS, CC)
    eye = jnp.broadcast_to(
        jnp.eye(NBINS, dtype=jnp.bfloat16), (TRB, NBINS, NBINS)
    )
    out_ref[...] = lax.dot_general(
        vb, eye, (((1,), (1,)), ((0,), (0,))),
        preferred_element_type=jnp.float32,
    )


def _fma_transpose(g4, w4):
    return pl.pallas_call(
        _fma_body,
        grid=(RR // TRB,),
        in_specs=[
            pl.BlockSpec((TRB * NBINS * 4, CC), lambda i: (i, 0)),
            pl.BlockSpec((TRB * NBINS, 4), lambda i: (i, 0)),
        ],
        out_specs=pl.BlockSpec((TRB, CC, NBINS), lambda i: (i, 0, 0)),
        out_shape=jax.ShapeDtypeStruct((RR, CC, NBINS), jnp.float32),
    )(g4, w4)


def kernel(features, proposals):
    idx4, w4 = _coords(proposals)
    table = features.transpose(0, 2, 3, 1).reshape(BB * HH * WW, CC)
    g4 = _sc_kernel(table, idx4)
    out3 = _fma_transpose(g4, w4)
    return out3.reshape(RR, CC, OUT, OUT)


# speedup vs baseline: 1.2468x; 1.2468x over previous
import functools

import jax
import jax.numpy as jnp
from jax import lax
from jax.experimental import pallas as pl
from jax.experimental.pallas import tpu as pltpu
from jax.experimental.pallas import tpu_sc as plsc

OUT = 7
SCALE = 6.5
BB, NN, CC, HH, WW = 2, 1000, 256, 128, 128
RR = BB * NN
RPAD = 2048
NBINS = OUT * OUT
NTOT = RPAD * NBINS
ROWS_PER_STEP = 32
GROWS = 4 * ROWS_PER_STEP
NWORKERS = 32
NSTEPS = NTOT // ROWS_PER_STEP
STEPS_PER_W = NSTEPS // NWORKERS
LANES = 16


def _coord_body(p_ref, idx_ref, w_ref):
    cx = p_ref[0:1, :]
    cy = p_ref[1:2, :]
    w = p_ref[2:3, :]
    h = p_ref[3:4, :]
    lane = lax.broadcasted_iota(jnp.int32, (1, RPAD), 1)
    in_range = (lane < RR).astype(jnp.float32)
    b = (lane >= NN).astype(jnp.int32)

    x1 = (cx - w * 0.5) * SCALE
    y1 = (cy - h * 0.5) * SCALE
    rsw = x1 - 0.5
    rsh = y1 - 0.5
    bin_w = (w * SCALE) / OUT
    bin_h = (h * SCALE) / OUT

    ii = lax.broadcasted_iota(jnp.int32, (NBINS, 1), 0)
    gi = (ii // OUT).astype(jnp.float32) + 0.5
    gj = (ii % OUT).astype(jnp.float32) + 0.5

    py = rsh + gi * bin_h
    px = rsw + gj * bin_w

    def interp(coord, size):
        valid = (coord >= -1.0) & (coord <= float(size))
        c = jnp.maximum(coord, 0.0)
        low = jnp.floor(c)
        cond = low >= float(size - 1)
        low = jnp.where(cond, float(size - 1), low)
        high = jnp.minimum(low + 1.0, float(size - 1))
        c = jnp.where(cond, float(size - 1), c)
        frac = c - low
        return low, high, frac, valid

    yl, yh, ly, vy = interp(py, HH)
    xl, xh, lx, vx = interp(px, WW)
    hy = 1.0 - ly
    hx = 1.0 - lx
    vf = (vy & vx).astype(jnp.float32) * in_range

    base = b * (HH * WW)
    yli = yl.astype(jnp.int32) * WW
    yhi = yh.astype(jnp.int32) * WW
    xli = xl.astype(jnp.int32)
    xhi = xh.astype(jnp.int32)

    idx_ref[...] = jnp.concatenate(
        [base + yli + xli, base + yli + xhi, base + yhi + xli, base + yhi + xhi],
        axis=0,
    )
    w_ref[...] = jnp.concatenate(
        [hy * hx * vf, hy * lx * vf, ly * hx * vf, ly * lx * vf], axis=0
    )


def _coords(proposals):
    prop = proposals.reshape(RR, 5).T
    propt = jnp.zeros((8, RPAD), jnp.float32).at[:5, :RR].set(prop)
    idx_all, w_all = pl.pallas_call(
        _coord_body,
        out_shape=(
            jax.ShapeDtypeStruct((4 * NBINS, RPAD), jnp.int32),
            jax.ShapeDtypeStruct((4 * NBINS, RPAD), jnp.float32),
        ),
    )(propt)
    idx4 = idx_all.reshape(4, NBINS, RPAD).transpose(2, 1, 0).reshape(NTOT * 4)
    w4 = w_all.reshape(4, NBINS, RPAD).transpose(2, 1, 0).reshape(NTOT * 4)
    return idx4, w4


def _sc_kernel(table, idx4, w4):
    mesh = plsc.VectorSubcoreMesh(core_axis_name="c", subcore_axis_name="s")
    LAST = STEPS_PER_W - 1
    PERW = STEPS_PER_W * GROWS

    @functools.partial(
        pl.kernel,
        mesh=mesh,
        out_type=jax.ShapeDtypeStruct((NTOT, CC), jnp.float32),
        scratch_types=[
            pltpu.VMEM((PERW,), jnp.int32),
            pltpu.VMEM((PERW,), jnp.float32),
            pltpu.VMEM((2, GROWS, CC), jnp.float32),
            pltpu.VMEM((2, ROWS_PER_STEP, CC), jnp.float32),
            pltpu.SemaphoreType.DMA((2, 2)),
        ],
    )
    def k(table_hbm, idx_hbm, w_hbm, out_hbm, idx_v, w_v, g_v, out_v, sems):
        wid = lax.axis_index("s") * 2 + lax.axis_index("c")
        base_step = wid * STEPS_PER_W

        pltpu.sync_copy(idx_hbm.at[pl.ds(base_step * GROWS, PERW)], idx_v)
        pltpu.sync_copy(w_hbm.at[pl.ds(base_step * GROWS, PERW)], w_v)

        def issue_gather(t, b):
            pltpu.make_async_copy(
                table_hbm.at[idx_v.at[pl.ds(t * GROWS, GROWS)]],
                g_v.at[b],
                sems.at[0, b],
            ).start()

        def wait_gather(b):
            pltpu.make_async_copy(
                table_hbm.at[idx_v.at[pl.ds(0, GROWS)]], g_v.at[b], sems.at[0, b]
            ).wait()

        def issue_store(t, b):
            row0 = (base_step + t) * ROWS_PER_STEP
            pltpu.make_async_copy(
                out_v.at[b], out_hbm.at[pl.ds(row0, ROWS_PER_STEP)], sems.at[1, b]
            ).start()

        def wait_store(b):
            pltpu.make_async_copy(
                out_v.at[b], out_hbm.at[pl.ds(0, ROWS_PER_STEP)], sems.at[1, b]
            ).wait()

        def compute(t, b):
            woff = t * GROWS

            def one_row(q, wq):
                w11, w12, w21, w22 = wq[0], wq[1], wq[2], wq[3]
                for cb in range(CC // LANES):
                    s = cb * LANES
                    acc = (
                        w11 * g_v[b, 4 * q, pl.ds(s, LANES)]
                        + w12 * g_v[b, 4 * q + 1, pl.ds(s, LANES)]
                    ) + (
                        w21 * g_v[b, 4 * q + 2, pl.ds(s, LANES)]
                        + w22 * g_v[b, 4 * q + 3, pl.ds(s, LANES)]
                    )
                    out_v[b, q, pl.ds(s, LANES)] = acc

            @pl.loop(0, ROWS_PER_STEP, step=4)
            def _(q):
                wv = w_v[pl.ds(woff + 4 * q, LANES)]
                for dq in range(4):
                    one_row(q + dq, [wv[4 * dq + k] for k in range(4)])

        issue_gather(0, 0)

        def body(t, b, nb):
            wait_gather(b)

            @pl.when(t < LAST)
            def _():
                issue_gather(t + 1, nb)

            @pl.when(t >= 2)
            def _():
                wait_store(b)

            compute(t, b)
            issue_store(t, b)

        @pl.loop(0, STEPS_PER_W // 2)
        def _(u):
            body(2 * u, 0, 1)
            body(2 * u + 1, 1, 0)

        wait_store(0)
        wait_store(1)

    return k(table, idx4, w4)


def kernel(features, proposals):
    idx4, wexp = _coords(proposals)
    table = features.transpose(0, 2, 3, 1).reshape(BB * HH * WW, CC)
    rows = _sc_kernel(table, idx4, wexp)
    out = rows.reshape(RPAD, NBINS, CC)[:RR]
    return out.transpose(0, 2, 1).reshape(RR, CC, OUT, OUT)


# speedup vs baseline: 1.3219x; 1.0602x over previous
import functools

import jax
import jax.numpy as jnp
from jax import lax
from jax.experimental import pallas as pl
from jax.experimental.pallas import tpu as pltpu
from jax.experimental.pallas import tpu_sc as plsc

OUT = 7
SCALE = 6.5
BB, NN, CC, HH, WW = 2, 1000, 256, 128, 128
RR = BB * NN
RPAD = 2048
NBINS = OUT * OUT
NTOT = RPAD * NBINS
ROWS_PER_STEP = 32
GROWS = 4 * ROWS_PER_STEP
NWORKERS = 32
NSTEPS = NTOT // ROWS_PER_STEP
STEPS_PER_W = NSTEPS // NWORKERS
LANES = 16


def _coord_body(p_ref, idx_ref, w_ref):
    cx = p_ref[0:1, :]
    cy = p_ref[1:2, :]
    w = p_ref[2:3, :]
    h = p_ref[3:4, :]
    lane = lax.broadcasted_iota(jnp.int32, (1, RPAD), 1)
    in_range = (lane < RR).astype(jnp.float32)
    b = (lane >= NN).astype(jnp.int32)

    x1 = (cx - w * 0.5) * SCALE
    y1 = (cy - h * 0.5) * SCALE
    rsw = x1 - 0.5
    rsh = y1 - 0.5
    bin_w = (w * SCALE) / OUT
    bin_h = (h * SCALE) / OUT

    ii = lax.broadcasted_iota(jnp.int32, (NBINS, 1), 0)
    gi = (ii // OUT).astype(jnp.float32) + 0.5
    gj = (ii % OUT).astype(jnp.float32) + 0.5

    py = rsh + gi * bin_h
    px = rsw + gj * bin_w

    def interp(coord, size):
        valid = (coord >= -1.0) & (coord <= float(size))
        c = jnp.maximum(coord, 0.0)
        low = jnp.floor(c)
        cond = low >= float(size - 1)
        low = jnp.where(cond, float(size - 1), low)
        high = jnp.minimum(low + 1.0, float(size - 1))
        c = jnp.where(cond, float(size - 1), c)
        frac = c - low
        return low, high, frac, valid

    yl, yh, ly, vy = interp(py, HH)
    xl, xh, lx, vx = interp(px, WW)
    hy = 1.0 - ly
    hx = 1.0 - lx
    vf = (vy & vx).astype(jnp.float32) * in_range

    base = b * (HH * WW)
    yli = yl.astype(jnp.int32) * WW
    yhi = yh.astype(jnp.int32) * WW
    xli = xl.astype(jnp.int32)
    xhi = xh.astype(jnp.int32)

    idx_ref[...] = jnp.concatenate(
        [base + yli + xli, base + yli + xhi, base + yhi + xli, base + yhi + xhi],
        axis=0,
    )
    w_ref[...] = jnp.concatenate(
        [hy * hx * vf, hy * lx * vf, ly * hx * vf, ly * lx * vf], axis=0
    )


def _coords(proposals):
    prop = proposals.reshape(RR, 5).T
    propt = jnp.zeros((8, RPAD), jnp.float32).at[:5, :RR].set(prop)
    idx_all, w_all = pl.pallas_call(
        _coord_body,
        out_shape=(
            jax.ShapeDtypeStruct((4 * NBINS, RPAD), jnp.int32),
            jax.ShapeDtypeStruct((4 * NBINS, RPAD), jnp.float32),
        ),
    )(propt)
    idx4 = idx_all.reshape(4, NBINS, RPAD).transpose(2, 1, 0).reshape(NTOT * 4)
    w4 = w_all.reshape(4, NBINS, RPAD).transpose(2, 1, 0).reshape(NTOT, 4)
    wexp = jnp.repeat(w4, LANES, axis=1)
    return idx4, wexp


def _sc_kernel(table, idx4, wexp):
    mesh = plsc.VectorSubcoreMesh(core_axis_name="c", subcore_axis_name="s")
    LAST = STEPS_PER_W - 1

    @functools.partial(
        pl.kernel,
        mesh=mesh,
        out_type=jax.ShapeDtypeStruct((NTOT, CC), jnp.float32),
        scratch_types=[
            pltpu.VMEM((2, GROWS), jnp.int32),
            pltpu.VMEM((2, ROWS_PER_STEP, 4 * LANES), jnp.float32),
            pltpu.VMEM((2, GROWS, CC), jnp.float32),
            pltpu.VMEM((2, ROWS_PER_STEP, CC), jnp.float32),
            pltpu.SemaphoreType.DMA((4, 2)),
        ],
    )
    def k(table_hbm, idx_hbm, w_hbm, out_hbm, idx_v, w_v, g_v, out_v, sems):
        wid = lax.axis_index("s") * 2 + lax.axis_index("c")
        base_step = wid * STEPS_PER_W

        def issue_idxw(t, b):
            row0 = (base_step + t) * ROWS_PER_STEP
            pltpu.make_async_copy(
                idx_hbm.at[pl.ds(row0 * 4, GROWS)], idx_v.at[b], sems.at[0, b]
            ).start()
            pltpu.make_async_copy(
                w_hbm.at[pl.ds(row0, ROWS_PER_STEP)], w_v.at[b], sems.at[1, b]
            ).start()

        def wait_idxw(b):
            pltpu.make_async_copy(
                idx_hbm.at[pl.ds(0, GROWS)], idx_v.at[b], sems.at[0, b]
            ).wait()
            pltpu.make_async_copy(
                w_hbm.at[pl.ds(0, ROWS_PER_STEP)], w_v.at[b], sems.at[1, b]
            ).wait()

        def issue_gather(b):
            pltpu.make_async_copy(
                table_hbm.at[idx_v.at[b]], g_v.at[b], sems.at[2, b]
            ).start()

        def wait_gather(b):
            pltpu.make_async_copy(
                table_hbm.at[idx_v.at[b]], g_v.at[b], sems.at[2, b]
            ).wait()

        def issue_store(t, b):
            row0 = (base_step + t) * ROWS_PER_STEP
            pltpu.make_async_copy(
                out_v.at[b], out_hbm.at[pl.ds(row0, ROWS_PER_STEP)], sems.at[3, b]
            ).start()

        def wait_store(b):
            pltpu.make_async_copy(
                out_v.at[b], out_hbm.at[pl.ds(0, ROWS_PER_STEP)], sems.at[3, b]
            ).wait()

        def compute(b):
            def one_row(q):
                w11 = w_v[b, q, pl.ds(0, LANES)]
                w12 = w_v[b, q, pl.ds(LANES, LANES)]
                w21 = w_v[b, q, pl.ds(2 * LANES, LANES)]
                w22 = w_v[b, q, pl.ds(3 * LANES, LANES)]
                for cb in range(CC // LANES):
                    s = cb * LANES
                    acc = (
                        w11 * g_v[b, 4 * q, pl.ds(s, LANES)]
                        + w12 * g_v[b, 4 * q + 1, pl.ds(s, LANES)]
                    ) + (
                        w21 * g_v[b, 4 * q + 2, pl.ds(s, LANES)]
                        + w22 * g_v[b, 4 * q + 3, pl.ds(s, LANES)]
                    )
                    out_v[b, q, pl.ds(s, LANES)] = acc

            @pl.loop(0, ROWS_PER_STEP, step=2)
            def _(q):
                one_row(q)
                one_row(q + 1)

        issue_idxw(0, 0)
        issue_idxw(1, 1)
        wait_idxw(0)
        issue_gather(0)

        def body(t, b, nb):
            @pl.when(t < LAST)
            def _():
                wait_idxw(nb)
                issue_gather(nb)

            wait_gather(b)

            @pl.when(t >= 2)
            def _():
                wait_store(b)

            compute(b)
            issue_store(t, b)

            @pl.when(t + 2 <= LAST)
            def _():
                issue_idxw(t + 2, b)

        @pl.loop(0, STEPS_PER_W // 2)
        def _(u):
            body(2 * u, 0, 1)
            body(2 * u + 1, 1, 0)

        wait_store(0)
        wait_store(1)

    return k(table, idx4, wexp)


def kernel(features, proposals):
    idx4, wexp = _coords(proposals)
    table = features.transpose(0, 2, 3, 1).reshape(BB * HH * WW, CC)
    rows = _sc_kernel(table, idx4, wexp)
    out = rows.reshape(RPAD, NBINS, CC)[:RR]
    return out.transpose(0, 2, 1).reshape(RR, CC, OUT, OUT)
